# baseline (speedup 1.0000x reference)
import functools

import jax
import jax.numpy as jnp
from jax import lax
from jax.experimental import pallas as pl
from jax.experimental.pallas import tpu as pltpu
from jax.experimental.pallas import tpu_sc as plsc

_N = 256
_NS = 1024
_NY = 64
_NX = 64
_D = 6
_A = 7

_NEG = -1e30


_NB = 8


def _sub_body(x_ref, m_ref, o_ref):
    x = x_ref[:, 0]
    mk = m_ref[0] > 0.0
    xm = jnp.where(mk[None], x, _NEG)
    mx = jnp.max(xm, axis=1, keepdims=True)
    e = jnp.where(mk[None], jnp.exp(x - mx), 0.0)
    s = jnp.sum(e, axis=1, keepdims=True)
    o_ref[:, 0] = e * (1.0 / s)


def _sub_policy_call(sub_logit, tmf):
    return pl.pallas_call(
        _sub_body,
        grid=(_D, _N // _NB),
        in_specs=[
            pl.BlockSpec((_NB, 1, _A, _NY, _NX), lambda d, nb: (nb, d, 0, 0, 0)),
            pl.BlockSpec((1, _A, _NY, _NX), lambda d, nb: (d, 0, 0, 0)),
        ],
        out_specs=pl.BlockSpec((_NB, 1, _A, _NY, _NX), lambda d, nb: (nb, d, 0, 0, 0)),
        out_shape=jax.ShapeDtypeStruct((_N, _D, _A, _NY, _NX), jnp.float32),
    )(sub_logit, tmf)


_NBD = 32


def _dip_body(x_ref, m_ref, o_ref):
    x = x_ref[...]
    mk = m_ref[...] > 0.0
    xm = jnp.where(mk, x, _NEG)
    mx = jnp.max(jnp.max(xm, axis=2, keepdims=True), axis=1, keepdims=True)
    e = jnp.where(mk, jnp.exp(x - mx), 0.0)
    s = jnp.sum(jnp.sum(e, axis=2, keepdims=True), axis=1, keepdims=True)
    o_ref[...] = e * (1.0 / s)


def _dip_policy_call(dip_logit, dmf):
    return pl.pallas_call(
        _dip_body,
        grid=(_N // _NBD,),
        in_specs=[
            pl.BlockSpec((_NBD, _NY, _NX), lambda i: (i, 0, 0)),
            pl.BlockSpec((_NBD, _NY, _NX), lambda i: (i, 0, 0)),
        ],
        out_specs=pl.BlockSpec((_NBD, _NY, _NX), lambda i: (i, 0, 0)),
        out_shape=jax.ShapeDtypeStruct((_N, _NY, _NX), jnp.float32),
    )(dip_logit, dmf)


_TILES = 32
_SAMP = _N * _NS
_PER_TILE = _SAMP // _TILES
_CHUNK = 128
_NCHUNK = _PER_TILE // _CHUNK
_GRP = _CHUNK // 16
_CELL = _NY * _NX


@functools.partial(
    pl.kernel,
    out_type=jax.ShapeDtypeStruct((_A * _SAMP,), jnp.float32),
    mesh=plsc.VectorSubcoreMesh(core_axis_name="c", subcore_axis_name="s"),
    scratch_types=(
        [pltpu.VMEM((_PER_TILE,), jnp.int32)] * 3
        + [pltpu.VMEM((_CHUNK,), jnp.int32)] * (_A + 1)
        + [pltpu.VMEM((_CHUNK,), jnp.float32)] * _A
        + [pltpu.VMEM((_CHUNK,), jnp.int32)]
        + [pltpu.VMEM((_PER_TILE,), jnp.float32)] * _A
        + [pltpu.SemaphoreType.DMA]
    ),
)
def _sc_sampled(logit_hbm, y_hbm, x_hbm, dr_hbm, mbits_hbm, out_hbm, *scr):
    y_v, x_v, dr_v = scr[0:3]
    idx_bufs = scr[3:3 + _A + 1]
    vals_bufs = scr[3 + _A + 1:3 + 2 * _A + 1]
    mvals_v = scr[3 + 2 * _A + 1]
    out_bufs = scr[3 + 2 * _A + 2:3 + 3 * _A + 2]
    sem = scr[3 + 3 * _A + 2]

    wid = lax.axis_index("s") * 2 + lax.axis_index("c")
    s0 = wid * _PER_TILE
    pltpu.sync_copy(y_hbm.at[pl.ds(s0, _PER_TILE)], y_v)
    pltpu.sync_copy(x_hbm.at[pl.ds(s0, _PER_TILE)], x_v)
    pltpu.sync_copy(dr_hbm.at[pl.ds(s0, _PER_TILE)], dr_v)

    def chunk_body(c, carry):
        n_val = wid * (_PER_TILE // _NS) + c // (_NS // _CHUNK)
        n_base = n_val * (_D * _A * _CELL)
        for g in range(_GRP):
            off = c * _CHUNK + g * 16
            yv = y_v[pl.ds(off, 16)]
            xv = x_v[pl.ds(off, 16)]
            dv = dr_v[pl.ds(off, 16)]
            cell = yv * _NX + xv
            b = n_base + dv * (_A * _CELL) + cell
            for a in range(_A):
                idx_bufs[a][pl.ds(g * 16, 16)] = b + a * _CELL
            idx_bufs[_A][pl.ds(g * 16, 16)] = dv * _CELL + cell
        cps = [pltpu.async_copy(logit_hbm.at[idx_bufs[a]], vals_bufs[a], sem)
               for a in range(_A)]
        cps.append(pltpu.async_copy(mbits_hbm.at[idx_bufs[_A]], mvals_v, sem))
        for cp in cps:
            cp.wait()
        for g in range(_GRP):
            off = c * _CHUNK + g * 16
            bits = mvals_v[pl.ds(g * 16, 16)]
            vs = [vals_bufs[a][pl.ds(g * 16, 16)] for a in range(_A)]
            mks = [((bits >> a) & 1) == 1 for a in range(_A)]
            xm = [jnp.where(m, v, _NEG) for m, v in zip(mks, vs)]
            mx = functools.reduce(jnp.maximum, xm)
            es = [jnp.where(m, jnp.exp(v - mx), 0.0) for m, v in zip(mks, vs)]
            ssum = functools.reduce(jnp.add, es)
            r = 1.0 / ssum
            for a in range(_A):
                out_bufs[a][pl.ds(off, 16)] = es[a] * r
        return carry

    lax.fori_loop(0, _NCHUNK, chunk_body, 0)
    for a in range(_A):
        pltpu.sync_copy(out_bufs[a], out_hbm.at[pl.ds(a * _SAMP + s0, _PER_TILE)])


def kernel(sub_logit, dip_logit, sub_pos_samples, dip_mask, transition_mask):
    tmf = transition_mask.astype(jnp.float32)
    dmf = dip_mask.astype(jnp.float32)

    sub_policy = _sub_policy_call(sub_logit, tmf)
    dip_policy = _dip_policy_call(dip_logit, dmf)

    sps = sub_pos_samples.astype(jnp.int32)
    y = sps[:, :, 0].reshape(-1)
    x = sps[:, :, 1].reshape(-1)
    dr = sps[:, :, 2].reshape(-1)
    tm_i = transition_mask.astype(jnp.int32)
    weights = (jnp.int32(1) << jnp.arange(_A, dtype=jnp.int32))[None, :, None, None]
    mbits = jnp.sum(tm_i * weights, axis=1).reshape(-1)

    sampled = _sc_sampled(sub_logit.reshape(-1), y, x, dr, mbits)
    sampled = sampled.reshape(_A, _N, _NS).transpose(1, 2, 0)
    return (sub_policy, sampled, dip_policy)


# speedup vs baseline: 3.2223x; 3.2223x over previous
import functools

import jax
import jax.numpy as jnp
from jax import lax
from jax.experimental import pallas as pl
from jax.experimental.pallas import tpu as pltpu
from jax.experimental.pallas import tpu_sc as plsc

_N = 256
_NS = 1024
_NY = 64
_NX = 64
_D = 6
_A = 7

_NEG = -1e30


_YB = 8


def _sub_body(x_ref, m_ref, o_ref):
    x = x_ref[0]
    mk = m_ref[0] != 0
    xm = jnp.where(mk, x, _NEG)
    mx = jnp.max(xm, axis=0, keepdims=True)
    e = jnp.where(mk, jnp.exp(x - mx), 0.0)
    s = jnp.sum(e, axis=0, keepdims=True)
    o_ref[0] = e * (1.0 / s)


def _sub_policy_call(slt, mask_b):
    blk = (1, _A, _YB, _NX, _N)
    return pl.pallas_call(
        _sub_body,
        grid=(_D, _NY // _YB),
        in_specs=[
            pl.BlockSpec(blk, lambda d, yb: (d, 0, yb, 0, 0)),
            pl.BlockSpec(blk, lambda d, yb: (d, 0, yb, 0, 0)),
        ],
        out_specs=pl.BlockSpec(blk, lambda d, yb: (d, 0, yb, 0, 0)),
        out_shape=jax.ShapeDtypeStruct((_D, _A, _NY, _NX, _N), jnp.float32),
    )(slt, mask_b)


def _dip_body(x_ref, m_ref, o_ref):
    x = x_ref[...]
    mk = m_ref[...] != 0
    xm = jnp.where(mk, x, _NEG)
    mx = jnp.max(jnp.max(xm, axis=1, keepdims=True), axis=0, keepdims=True)
    e = jnp.where(mk, jnp.exp(x - mx), 0.0)
    s = jnp.sum(jnp.sum(e, axis=1, keepdims=True), axis=0, keepdims=True)
    o_ref[...] = e * (1.0 / s)


def _dip_policy_call(dlt, dmt):
    return pl.pallas_call(
        _dip_body,
        out_shape=jax.ShapeDtypeStruct((_NY, _NX, _N), jnp.float32),
    )(dlt, dmt)


_TILES = 32
_SAMP = _N * _NS
_PER_TILE = _SAMP // _TILES
_CHUNK = 128
_NCHUNK = _PER_TILE // _CHUNK
_GRP = _CHUNK // 16
_CELL = _NY * _NX
_CA = _CELL * _N


@functools.partial(
    pl.kernel,
    out_type=jax.ShapeDtypeStruct((_A * _SAMP,), jnp.float32),
    mesh=plsc.VectorSubcoreMesh(core_axis_name="c", subcore_axis_name="s"),
    scratch_types=(
        [pltpu.VMEM((_PER_TILE,), jnp.int32)] * 3
        + [pltpu.VMEM((_CHUNK,), jnp.int32)] * (_A + 1)
        + [pltpu.VMEM((_CHUNK,), jnp.float32)] * _A
        + [pltpu.VMEM((_CHUNK,), jnp.int32)]
        + [pltpu.VMEM((_PER_TILE,), jnp.float32)] * _A
        + [pltpu.SemaphoreType.DMA]
    ),
)
def _sc_sampled(logit_hbm, y_hbm, x_hbm, dr_hbm, mbits_hbm, out_hbm, *scr):
    y_v, x_v, dr_v = scr[0:3]
    idx_bufs = scr[3:3 + _A + 1]
    vals_bufs = scr[3 + _A + 1:3 + 2 * _A + 1]
    mvals_v = scr[3 + 2 * _A + 1]
    out_bufs = scr[3 + 2 * _A + 2:3 + 3 * _A + 2]
    sem = scr[3 + 3 * _A + 2]

    wid = lax.axis_index("s") * 2 + lax.axis_index("c")
    s0 = wid * _PER_TILE
    pltpu.sync_copy(y_hbm.at[pl.ds(s0, _PER_TILE)], y_v)
    pltpu.sync_copy(x_hbm.at[pl.ds(s0, _PER_TILE)], x_v)
    pltpu.sync_copy(dr_hbm.at[pl.ds(s0, _PER_TILE)], dr_v)

    def chunk_body(c, carry):
        n_val = wid * (_PER_TILE // _NS) + c // (_NS // _CHUNK)
        for g in range(_GRP):
            off = c * _CHUNK + g * 16
            yv = y_v[pl.ds(off, 16)]
            xv = x_v[pl.ds(off, 16)]
            dv = dr_v[pl.ds(off, 16)]
            b = dv * (_A * _CA) + yv * (_NX * _N) + xv * _N + n_val
            for a in range(_A):
                idx_bufs[a][pl.ds(g * 16, 16)] = b + a * _CA
            idx_bufs[_A][pl.ds(g * 16, 16)] = dv * _CELL + yv * _NX + xv
        cps = [pltpu.async_copy(logit_hbm.at[idx_bufs[a]], vals_bufs[a], sem)
               for a in range(_A)]
        cps.append(pltpu.async_copy(mbits_hbm.at[idx_bufs[_A]], mvals_v, sem))
        for cp in cps:
            cp.wait()
        for g in range(_GRP):
            off = c * _CHUNK + g * 16
            bits = mvals_v[pl.ds(g * 16, 16)]
            vs = [vals_bufs[a][pl.ds(g * 16, 16)] for a in range(_A)]
            mks = [((bits >> a) & 1) == 1 for a in range(_A)]
            xm = [jnp.where(m, v, _NEG) for m, v in zip(mks, vs)]
            mx = functools.reduce(jnp.maximum, xm)
            es = [jnp.where(m, jnp.exp(v - mx), 0.0) for m, v in zip(mks, vs)]
            ssum = functools.reduce(jnp.add, es)
            r = 1.0 / ssum
            for a in range(_A):
                out_bufs[a][pl.ds(off, 16)] = es[a] * r
        return carry

    lax.fori_loop(0, _NCHUNK, chunk_body, 0)
    for a in range(_A):
        pltpu.sync_copy(out_bufs[a], out_hbm.at[pl.ds(a * _SAMP + s0, _PER_TILE)])


def kernel(sub_logit, dip_logit, sub_pos_samples, dip_mask, transition_mask):
    slt = jnp.transpose(sub_logit, (1, 2, 3, 4, 0))
    dlt = jnp.transpose(dip_logit, (1, 2, 0))
    dmt = jnp.transpose(dip_mask, (1, 2, 0)).astype(jnp.uint8)
    mask_b = jnp.broadcast_to(
        transition_mask[..., None].astype(jnp.uint8), (_D, _A, _NY, _NX, _N))

    sub_policy_t = _sub_policy_call(slt, mask_b)
    dip_policy_t = _dip_policy_call(dlt, dmt)
    sub_policy = jnp.transpose(sub_policy_t, (4, 0, 1, 2, 3))
    dip_policy = jnp.transpose(dip_policy_t, (2, 0, 1))

    sps = sub_pos_samples.astype(jnp.int32)
    y = sps[:, :, 0].reshape(-1)
    x = sps[:, :, 1].reshape(-1)
    dr = sps[:, :, 2].reshape(-1)
    tm_i = transition_mask.astype(jnp.int32)
    weights = (jnp.int32(1) << jnp.arange(_A, dtype=jnp.int32))[None, :, None, None]
    mbits = jnp.sum(tm_i * weights, axis=1).reshape(-1)

    flt = slt.reshape(-1)
    sampled = _sc_sampled(flt, y, x, dr, mbits)
    sampled = sampled.reshape(_A, _N, _NS).transpose(1, 2, 0)
    return (sub_policy, sampled, dip_policy)


# speedup vs baseline: 3.3413x; 1.0369x over previous
import functools

import jax
import jax.numpy as jnp
from jax import lax
from jax.experimental import pallas as pl
from jax.experimental.pallas import tpu as pltpu
from jax.experimental.pallas import tpu_sc as plsc

_N = 256
_NS = 1024
_NY = 64
_NX = 64
_D = 6
_A = 7

_NEG = -1e30


_YB = 8


def _sub_body(x_ref, m_ref, o_ref):
    x = x_ref[0]
    mk = m_ref[0] != 0
    xm = jnp.where(mk, x, _NEG)
    mx = jnp.max(xm, axis=0, keepdims=True)
    e = jnp.where(mk, jnp.exp(x - mx), 0.0)
    s = jnp.sum(e, axis=0, keepdims=True)
    o_ref[0] = e * (1.0 / s)


def _sub_policy_call(slt, mask_b):
    blk = (1, _A, _YB, _NX, _N)
    return pl.pallas_call(
        _sub_body,
        grid=(_D, _NY // _YB),
        in_specs=[
            pl.BlockSpec(blk, lambda d, yb: (d, 0, yb, 0, 0)),
            pl.BlockSpec(blk, lambda d, yb: (d, 0, yb, 0, 0)),
        ],
        out_specs=pl.BlockSpec(blk, lambda d, yb: (d, 0, yb, 0, 0)),
        out_shape=jax.ShapeDtypeStruct((_D, _A, _NY, _NX, _N), jnp.float32),
    )(slt, mask_b)


def _dip_body(x_ref, m_ref, o_ref):
    x = x_ref[...]
    mk = m_ref[...] != 0
    xm = jnp.where(mk, x, _NEG)
    mx = jnp.max(jnp.max(xm, axis=1, keepdims=True), axis=0, keepdims=True)
    e = jnp.where(mk, jnp.exp(x - mx), 0.0)
    s = jnp.sum(jnp.sum(e, axis=1, keepdims=True), axis=0, keepdims=True)
    o_ref[...] = e * (1.0 / s)


def _dip_policy_call(dlt, dmt):
    return pl.pallas_call(
        _dip_body,
        out_shape=jax.ShapeDtypeStruct((_NY, _NX, _N), jnp.float32),
    )(dlt, dmt)


_TILES = 32
_SAMP = _N * _NS
_PER_TILE = _SAMP // _TILES
_CHUNK = 128
_NCHUNK = _PER_TILE // _CHUNK
_GRP = _CHUNK // 16
_CELL = _NY * _NX
_CA = _CELL * _N


@functools.partial(
    pl.kernel,
    out_type=jax.ShapeDtypeStruct((_A * _SAMP,), jnp.float32),
    mesh=plsc.VectorSubcoreMesh(core_axis_name="c", subcore_axis_name="s"),
    scratch_types=(
        [pltpu.VMEM((_PER_TILE,), jnp.int32)] * 3
        + [pltpu.VMEM((_CHUNK,), jnp.int32)] * (_A + 1)
        + [pltpu.VMEM((_CHUNK,), jnp.float32)] * _A
        + [pltpu.VMEM((_CHUNK,), jnp.int32)]
        + [pltpu.VMEM((_CHUNK,), jnp.int32)] * (_A + 1)
        + [pltpu.VMEM((_CHUNK,), jnp.float32)] * _A
        + [pltpu.VMEM((_CHUNK,), jnp.int32)]
        + [pltpu.VMEM((_PER_TILE,), jnp.float32)] * _A
        + [pltpu.SemaphoreType.DMA] * 2
    ),
)
def _sc_sampled(logit_hbm, y_hbm, x_hbm, dr_hbm, mbits_hbm, out_hbm, *scr):
    _SET = 2 * _A + 2
    y_v, x_v, dr_v = scr[0:3]
    sets = []
    for si in range(2):
        b = 3 + si * _SET
        sets.append((scr[b:b + _A + 1],
                     scr[b + _A + 1:b + 2 * _A + 1],
                     scr[b + 2 * _A + 1]))
    out_bufs = scr[3 + 2 * _SET:3 + 2 * _SET + _A]
    sems = scr[3 + 2 * _SET + _A:3 + 2 * _SET + _A + 2]

    wid = lax.axis_index("s") * 2 + lax.axis_index("c")
    s0 = wid * _PER_TILE
    pltpu.sync_copy(y_hbm.at[pl.ds(s0, _PER_TILE)], y_v)
    pltpu.sync_copy(x_hbm.at[pl.ds(s0, _PER_TILE)], x_v)
    pltpu.sync_copy(dr_hbm.at[pl.ds(s0, _PER_TILE)], dr_v)

    def build_fire(c, si):
        idxs, vals, mvals = sets[si]
        n_val = wid * (_PER_TILE // _NS) + c // (_NS // _CHUNK)
        for g in range(_GRP):
            off = c * _CHUNK + g * 16
            yv = y_v[pl.ds(off, 16)]
            xv = x_v[pl.ds(off, 16)]
            dv = dr_v[pl.ds(off, 16)]
            b = dv * (_A * _CA) + yv * (_NX * _N) + xv * _N + n_val
            for a in range(_A):
                idxs[a][pl.ds(g * 16, 16)] = b + a * _CA
            idxs[_A][pl.ds(g * 16, 16)] = dv * _CELL + yv * _NX + xv
        for a in range(_A):
            pltpu.async_copy(logit_hbm.at[idxs[a]], vals[a], sems[si])
        pltpu.async_copy(mbits_hbm.at[idxs[_A]], mvals, sems[si])

    def drain(si):
        idxs, vals, mvals = sets[si]
        for a in range(_A):
            pltpu.make_async_copy(logit_hbm.at[idxs[a]], vals[a], sems[si]).wait()
        pltpu.make_async_copy(mbits_hbm.at[idxs[_A]], mvals, sems[si]).wait()

    def compute(c, si):
        _, vals, mvals = sets[si]
        for g in range(_GRP):
            off = c * _CHUNK + g * 16
            bits = mvals[pl.ds(g * 16, 16)]
            vs = [vals[a][pl.ds(g * 16, 16)] for a in range(_A)]
            mks = [((bits >> a) & 1) == 1 for a in range(_A)]
            xm = [jnp.where(m, v, _NEG) for m, v in zip(mks, vs)]
            mx = functools.reduce(jnp.maximum, xm)
            es = [jnp.exp(v - mx) for v in xm]
            ssum = functools.reduce(jnp.add, es)
            r = 1.0 / ssum
            for a in range(_A):
                out_bufs[a][pl.ds(off, 16)] = es[a] * r

    build_fire(0, 0)

    def pair_body(i, carry):
        c0 = 2 * i
        build_fire(c0 + 1, 1)
        drain(0)
        compute(c0, 0)

        @pl.when(i < _NCHUNK // 2 - 1)
        def _():
            build_fire(c0 + 2, 0)

        drain(1)
        compute(c0 + 1, 1)
        return carry

    lax.fori_loop(0, _NCHUNK // 2, pair_body, 0)
    for a in range(_A):
        pltpu.sync_copy(out_bufs[a], out_hbm.at[pl.ds(a * _SAMP + s0, _PER_TILE)])


def kernel(sub_logit, dip_logit, sub_pos_samples, dip_mask, transition_mask):
    slt = jnp.transpose(sub_logit, (1, 2, 3, 4, 0))
    dlt = jnp.transpose(dip_logit, (1, 2, 0))
    dmt = jnp.transpose(dip_mask, (1, 2, 0)).astype(jnp.uint8)
    mask_b = jnp.broadcast_to(
        transition_mask[..., None].astype(jnp.uint8), (_D, _A, _NY, _NX, _N))

    sub_policy_t = _sub_policy_call(slt, mask_b)
    dip_policy_t = _dip_policy_call(dlt, dmt)
    sub_policy = jnp.transpose(sub_policy_t, (4, 0, 1, 2, 3))
    dip_policy = jnp.transpose(dip_policy_t, (2, 0, 1))

    sps = sub_pos_samples.astype(jnp.int32)
    y = sps[:, :, 0].reshape(-1)
    x = sps[:, :, 1].reshape(-1)
    dr = sps[:, :, 2].reshape(-1)
    tm_i = transition_mask.astype(jnp.int32)
    weights = (jnp.int32(1) << jnp.arange(_A, dtype=jnp.int32))[None, :, None, None]
    mbits = jnp.sum(tm_i * weights, axis=1).reshape(-1)

    flt = slt.reshape(-1)
    sampled = _sc_sampled(flt, y, x, dr, mbits)
    sampled = sampled.reshape(_A, _N, _NS).transpose(1, 2, 0)
    return (sub_policy, sampled, dip_policy)


# speedup vs baseline: 3.3778x; 1.0109x over previous
import functools

import jax
import jax.numpy as jnp
from jax import lax
from jax.experimental import pallas as pl
from jax.experimental.pallas import tpu as pltpu
from jax.experimental.pallas import tpu_sc as plsc

_N = 256
_NS = 1024
_NY = 64
_NX = 64
_D = 6
_A = 7

_NEG = -1e30


_YB = 8


def _sub_body(x_ref, m_ref, o_ref):
    x = x_ref[0]
    mk = m_ref[0] != 0
    xm = jnp.where(mk, x, _NEG)
    mx = jnp.max(xm, axis=0, keepdims=True)
    e = jnp.where(mk, jnp.exp(x - mx), 0.0)
    s = jnp.sum(e, axis=0, keepdims=True)
    o_ref[0] = e * (1.0 / s)


def _sub_policy_call(slt, mask_b):
    blk = (1, _A, _YB, _NX, _N)
    return pl.pallas_call(
        _sub_body,
        grid=(_D, _NY // _YB),
        in_specs=[
            pl.BlockSpec(blk, lambda d, yb: (d, 0, yb, 0, 0)),
            pl.BlockSpec(blk, lambda d, yb: (d, 0, yb, 0, 0)),
        ],
        out_specs=pl.BlockSpec(blk, lambda d, yb: (d, 0, yb, 0, 0)),
        out_shape=jax.ShapeDtypeStruct((_D, _A, _NY, _NX, _N), jnp.float32),
    )(slt, mask_b)


def _dip_body(x_ref, m_ref, o_ref):
    x = x_ref[...]
    mk = m_ref[...] != 0
    xm = jnp.where(mk, x, _NEG)
    mx = jnp.max(jnp.max(xm, axis=1, keepdims=True), axis=0, keepdims=True)
    e = jnp.where(mk, jnp.exp(x - mx), 0.0)
    s = jnp.sum(jnp.sum(e, axis=1, keepdims=True), axis=0, keepdims=True)
    o_ref[...] = e * (1.0 / s)


def _dip_policy_call(dlt, dmt):
    return pl.pallas_call(
        _dip_body,
        out_shape=jax.ShapeDtypeStruct((_NY, _NX, _N), jnp.float32),
    )(dlt, dmt)


_TILES = 32
_SAMP = _N * _NS
_PER_TILE = _SAMP // _TILES
_CHUNK = 128
_NCHUNK = _PER_TILE // _CHUNK
_GRP = _CHUNK // 16
_CELL = _NY * _NX
_CA = _CELL * _N


@functools.partial(
    pl.kernel,
    out_type=jax.ShapeDtypeStruct((_A * _SAMP,), jnp.float32),
    mesh=plsc.VectorSubcoreMesh(core_axis_name="c", subcore_axis_name="s"),
    scratch_types=(
        [pltpu.VMEM((_PER_TILE,), jnp.int32)] * 3
        + [pltpu.VMEM((_CHUNK,), jnp.int32)] * (_A + 1)
        + [pltpu.VMEM((_CHUNK,), jnp.float32)] * _A
        + [pltpu.VMEM((_CHUNK,), jnp.int32)]
        + [pltpu.VMEM((_CHUNK,), jnp.int32)] * (_A + 1)
        + [pltpu.VMEM((_CHUNK,), jnp.float32)] * _A
        + [pltpu.VMEM((_CHUNK,), jnp.int32)]
        + [pltpu.VMEM((_PER_TILE,), jnp.float32)] * _A
        + [pltpu.SemaphoreType.DMA] * 2
    ),
)
def _sc_sampled(logit_hbm, y_hbm, x_hbm, dr_hbm, mbits_hbm, out_hbm, *scr):
    _SET = 2 * _A + 2
    y_v, x_v, dr_v = scr[0:3]
    sets = []
    for si in range(2):
        b = 3 + si * _SET
        sets.append((scr[b:b + _A + 1],
                     scr[b + _A + 1:b + 2 * _A + 1],
                     scr[b + 2 * _A + 1]))
    out_bufs = scr[3 + 2 * _SET:3 + 2 * _SET + _A]
    sems = scr[3 + 2 * _SET + _A:3 + 2 * _SET + _A + 2]

    wid = lax.axis_index("s") * 2 + lax.axis_index("c")
    s0 = wid * _PER_TILE
    pltpu.sync_copy(y_hbm.at[pl.ds(s0, _PER_TILE)], y_v)
    pltpu.sync_copy(x_hbm.at[pl.ds(s0, _PER_TILE)], x_v)
    pltpu.sync_copy(dr_hbm.at[pl.ds(s0, _PER_TILE)], dr_v)

    def build_fire(c, si):
        idxs, vals, mvals = sets[si]
        n_val = wid * (_PER_TILE // _NS) + c // (_NS // _CHUNK)
        for g in range(_GRP):
            off = c * _CHUNK + g * 16
            yv = y_v[pl.ds(off, 16)]
            xv = x_v[pl.ds(off, 16)]
            dv = dr_v[pl.ds(off, 16)]
            b = dv * (_A * _CA) + yv * (_NX * _N) + xv * _N + n_val
            for a in range(_A):
                idxs[a][pl.ds(g * 16, 16)] = b + a * _CA
            idxs[_A][pl.ds(g * 16, 16)] = dv * _CELL + yv * _NX + xv
        for a in range(_A):
            pltpu.async_copy(logit_hbm.at[idxs[a]], vals[a], sems[si])
        pltpu.async_copy(mbits_hbm.at[idxs[_A]], mvals, sems[si])

    def drain(si):
        idxs, vals, mvals = sets[si]
        for a in range(_A):
            pltpu.make_async_copy(logit_hbm.at[idxs[a]], vals[a], sems[si]).wait()
        pltpu.make_async_copy(mbits_hbm.at[idxs[_A]], mvals, sems[si]).wait()

    def compute(c, si):
        _, vals, mvals = sets[si]
        for g in range(_GRP):
            off = c * _CHUNK + g * 16
            bits = mvals[pl.ds(g * 16, 16)]
            vs = [vals[a][pl.ds(g * 16, 16)] for a in range(_A)]
            mks = [((bits >> a) & 1) == 1 for a in range(_A)]
            xm = [jnp.where(m, v, _NEG) for m, v in zip(mks, vs)]
            mx = functools.reduce(jnp.maximum, xm)
            es = [jnp.exp(v - mx) for v in xm]
            ssum = functools.reduce(jnp.add, es)
            r = 1.0 / ssum
            for a in range(_A):
                out_bufs[a][pl.ds(off, 16)] = es[a] * r

    build_fire(0, 0)

    def pair_body(i, carry):
        c0 = 2 * i
        build_fire(c0 + 1, 1)
        drain(0)
        compute(c0, 0)

        @pl.when(i < _NCHUNK // 2 - 1)
        def _():
            build_fire(c0 + 2, 0)

        drain(1)
        compute(c0 + 1, 1)
        return carry

    lax.fori_loop(0, _NCHUNK // 2, pair_body, 0)
    for a in range(_A):
        pltpu.sync_copy(out_bufs[a], out_hbm.at[pl.ds(a * _SAMP + s0, _PER_TILE)])


def kernel(sub_logit, dip_logit, sub_pos_samples, dip_mask, transition_mask):
    slt = jnp.transpose(sub_logit, (1, 2, 3, 4, 0))
    dlt = jnp.transpose(dip_logit, (1, 2, 0))
    dmt = jnp.transpose(dip_mask, (1, 2, 0)).astype(jnp.uint8)
    mask_b = jnp.broadcast_to(
        transition_mask[..., None].astype(jnp.uint8), (_D, _A, _NY, _NX, _N))

    sub_policy_t = _sub_policy_call(slt, mask_b)
    dip_policy_t = _dip_policy_call(dlt, dmt)
    sub_policy = jnp.transpose(sub_policy_t, (4, 0, 1, 2, 3))
    dip_policy = jnp.transpose(dip_policy_t, (2, 0, 1))

    sps = sub_pos_samples.astype(jnp.int32)
    y = sps[:, :, 0].reshape(-1)
    x = sps[:, :, 1].reshape(-1)
    dr = sps[:, :, 2].reshape(-1)
    tm_i = transition_mask.astype(jnp.int32)
    weights = (jnp.int32(1) << jnp.arange(_A, dtype=jnp.int32))[None, :, None, None]
    mbits = jnp.sum(tm_i * weights, axis=1).reshape(-1)
    mbits, _ = lax.optimization_barrier((mbits, mask_b))

    flt = slt.reshape(-1)
    sampled = _sc_sampled(flt, y, x, dr, mbits)
    sampled = sampled.reshape(_A, _N, _NS).transpose(1, 2, 0)
    return (sub_policy, sampled, dip_policy)
